# baseline (speedup 1.0000x reference)
import jax
import jax.numpy as jnp
from jax.experimental import pallas as pl
from jax.experimental.pallas import tpu as pltpu

QUEUE = 65536
DIM = 128
BATCH = 4096
NCH = QUEUE // BATCH
LROWS = BATCH // 128


def _win_kernel(s_ref, qe_ref, ql_ref, emb_ref, lab_ref, oe_ref, ol_ref):
    del s_ref, qe_ref, ql_ref
    x = emb_ref[...]
    n = jnp.sqrt(jnp.sum(x * x, axis=1, keepdims=True))
    oe_ref[...] = x / jnp.maximum(n, 1e-12)
    ol_ref[...] = lab_ref[...]


def kernel(embeddings, labels, queue_embeds, queue_labels, queue_ptr):
    ldtype = queue_labels.dtype
    ql2 = queue_labels.reshape(QUEUE // 128, 128)
    lab2 = labels.astype(ldtype).reshape(LROWS, 128)
    s_blk = jnp.reshape(
        jax.lax.rem(queue_ptr.astype(jnp.int32) // BATCH, NCH), (1,)
    )

    grid_spec = pltpu.PrefetchScalarGridSpec(
        num_scalar_prefetch=1,
        grid=(1,),
        in_specs=[
            pl.BlockSpec(memory_space=pl.ANY),
            pl.BlockSpec(memory_space=pl.ANY),
            pl.BlockSpec((BATCH, DIM), lambda i, s: (0, 0)),
            pl.BlockSpec((LROWS, 128), lambda i, s: (0, 0)),
        ],
        out_specs=[
            pl.BlockSpec((BATCH, DIM), lambda i, s: (s[0], 0)),
            pl.BlockSpec((LROWS, 128), lambda i, s: (s[0], 0)),
        ],
    )

    new_qe, new_ql2 = pl.pallas_call(
        _win_kernel,
        grid_spec=grid_spec,
        out_shape=[
            jax.ShapeDtypeStruct((QUEUE, DIM), queue_embeds.dtype),
            jax.ShapeDtypeStruct((QUEUE // 128, 128), ldtype),
        ],
        input_output_aliases={1: 0, 2: 1},
    )(s_blk, queue_embeds, ql2, embeddings, lab2)

    new_ptr = ((queue_ptr + BATCH) % QUEUE).astype(queue_ptr.dtype)
    return (new_qe, new_ql2.reshape(QUEUE), new_ptr)


# speedup vs baseline: 1.1453x; 1.1453x over previous
import jax
import jax.numpy as jnp
from jax.experimental import pallas as pl
from jax.experimental.pallas import tpu as pltpu

QUEUE = 65536
DIM = 128
BATCH = 4096
BS = 8192
NB = QUEUE // BS
WPB = BS // BATCH
LBS = BS // 128
LW = BATCH // 128


def _enqueue_kernel(s_ref, qe_ref, ql_ref, emb_ref, lab_ref, oe_ref, ol_ref):
    k = pl.program_id(0)
    s = s_ref[0]
    blk = s // WPB
    sub = jax.lax.rem(s, WPB)

    oe_ref[...] = qe_ref[...]
    ol_ref[...] = ql_ref[...]

    @pl.when(k == blk)
    def _():
        x = emb_ref[...]
        n = jnp.sqrt(jnp.sum(x * x, axis=1, keepdims=True))
        oe_ref[pl.ds(sub * BATCH, BATCH), :] = x / jnp.maximum(n, 1e-12)
        ol_ref[pl.ds(sub * LW, LW), :] = lab_ref[...]


def kernel(embeddings, labels, queue_embeds, queue_labels, queue_ptr):
    ldtype = queue_labels.dtype
    ql2 = queue_labels.reshape(QUEUE // 128, 128)
    lab2 = labels.astype(ldtype).reshape(LW, 128)
    s_blk = jnp.reshape(
        jax.lax.rem(queue_ptr.astype(jnp.int32) // BATCH, QUEUE // BATCH), (1,)
    )

    grid_spec = pltpu.PrefetchScalarGridSpec(
        num_scalar_prefetch=1,
        grid=(NB,),
        in_specs=[
            pl.BlockSpec((BS, DIM), lambda k, s: (k, 0)),
            pl.BlockSpec((LBS, 128), lambda k, s: (k, 0)),
            pl.BlockSpec((BATCH, DIM), lambda k, s: (0, 0)),
            pl.BlockSpec((LW, 128), lambda k, s: (0, 0)),
        ],
        out_specs=[
            pl.BlockSpec((BS, DIM), lambda k, s: (k, 0)),
            pl.BlockSpec((LBS, 128), lambda k, s: (k, 0)),
        ],
    )

    new_qe, new_ql2 = pl.pallas_call(
        _enqueue_kernel,
        grid_spec=grid_spec,
        out_shape=[
            jax.ShapeDtypeStruct((QUEUE, DIM), queue_embeds.dtype),
            jax.ShapeDtypeStruct((QUEUE // 128, 128), ldtype),
        ],
    )(s_blk, queue_embeds, ql2, embeddings, lab2)

    new_ptr = ((queue_ptr + BATCH) % QUEUE).astype(queue_ptr.dtype)
    return (new_qe, new_ql2.reshape(QUEUE), new_ptr)


# speedup vs baseline: 1.2200x; 1.0653x over previous
import jax
import jax.numpy as jnp
from jax.experimental import pallas as pl
from jax.experimental.pallas import tpu as pltpu

QUEUE = 65536
DIM = 128
BATCH = 4096
BS = 16384
NB = QUEUE // BS
WPB = BS // BATCH
LBS = BS // 128
LW = BATCH // 128


def _enqueue_kernel(s_ref, qe_ref, ql_ref, emb_ref, lab_ref, oe_ref, ol_ref):
    k = pl.program_id(0)
    s = s_ref[0]
    blk = s // WPB
    sub = jax.lax.rem(s, WPB)

    oe_ref[...] = qe_ref[...]
    ol_ref[...] = ql_ref[...]

    @pl.when(k == blk)
    def _():
        x = emb_ref[...]
        n = jnp.sqrt(jnp.sum(x * x, axis=1, keepdims=True))
        oe_ref[pl.ds(sub * BATCH, BATCH), :] = x / jnp.maximum(n, 1e-12)
        ol_ref[pl.ds(sub * LW, LW), :] = lab_ref[...]


def kernel(embeddings, labels, queue_embeds, queue_labels, queue_ptr):
    ldtype = queue_labels.dtype
    ql2 = queue_labels.reshape(QUEUE // 128, 128)
    lab2 = labels.astype(ldtype).reshape(LW, 128)
    s_blk = jnp.reshape(
        jax.lax.rem(queue_ptr.astype(jnp.int32) // BATCH, QUEUE // BATCH), (1,)
    )

    grid_spec = pltpu.PrefetchScalarGridSpec(
        num_scalar_prefetch=1,
        grid=(NB,),
        in_specs=[
            pl.BlockSpec((BS, DIM), lambda k, s: (k, 0)),
            pl.BlockSpec((LBS, 128), lambda k, s: (k, 0)),
            pl.BlockSpec((BATCH, DIM), lambda k, s: (0, 0)),
            pl.BlockSpec((LW, 128), lambda k, s: (0, 0)),
        ],
        out_specs=[
            pl.BlockSpec((BS, DIM), lambda k, s: (k, 0)),
            pl.BlockSpec((LBS, 128), lambda k, s: (k, 0)),
        ],
    )

    new_qe, new_ql2 = pl.pallas_call(
        _enqueue_kernel,
        grid_spec=grid_spec,
        out_shape=[
            jax.ShapeDtypeStruct((QUEUE, DIM), queue_embeds.dtype),
            jax.ShapeDtypeStruct((QUEUE // 128, 128), ldtype),
        ],
    )(s_blk, queue_embeds, ql2, embeddings, lab2)

    new_ptr = ((queue_ptr + BATCH) % QUEUE).astype(queue_ptr.dtype)
    return (new_qe, new_ql2.reshape(QUEUE), new_ptr)
